# baseline (speedup 1.0000x reference)
import functools

import jax
import jax.numpy as jnp
from jax import lax
from jax.experimental import pallas as pl
from jax.experimental.pallas import tpu as pltpu
from jax.experimental.pallas import tpu_sc as plsc

_CHUNK = 80
_NB = 4
_NI = 8


def _sc_segment_sum(edges, x):
    n, d = x.shape
    num_chunks = edges.shape[0]
    info = plsc.get_sparse_core_info()
    nc, ns = info.num_cores, info.num_subcores
    nw = nc * ns
    rows_per_tile = (n // ns) // 8 * 8
    rem_rows = n - rows_per_tile * ns
    zrows = 16
    assert rows_per_tile % zrows == 0 and rem_rows % zrows == 0
    assert _CHUNK >= zrows

    mesh = plsc.VectorSubcoreMesh(core_axis_name="c", subcore_axis_name="s")

    @functools.partial(
        pl.kernel,
        out_type=jax.ShapeDtypeStruct((nc, n, d), jnp.float32),
        mesh=mesh,
        scratch_types=[
            pltpu.VMEM_SHARED((n, d), jnp.float32),
            pltpu.VMEM((_NI, 2, _CHUNK), jnp.int32),
            pltpu.VMEM((_NB, _CHUNK, d), jnp.float32),
            pltpu.SemaphoreType.DMA((_NI,)),
            pltpu.SemaphoreType.DMA((_NB,)),
            pltpu.SemaphoreType.DMA((_NB,)),
            pltpu.SemaphoreType.DMA,
        ],
    )
    def seg_sum(edges_hbm, x_hbm, out_hbm, acc, ij, rows,
                sem_e, sem_g, sem_s, zsem):
        cid = lax.axis_index("c")
        sid = lax.axis_index("s")
        wid = sid * nc + cid
        row0 = sid * rows_per_tile

        my_chunks = (num_chunks - wid + nw - 1) // nw

        def fetch_idx(i, b):
            pltpu.async_copy(edges_hbm.at[wid + i * nw], ij.at[b],
                             sem_e.at[b])

        def issue_gather(ib, rb):
            pltpu.async_copy(x_hbm.at[ij.at[ib, 0]], rows.at[rb],
                             sem_g.at[rb])

        def wait_idx(b):
            pltpu.make_async_copy(edges_hbm.at[0], ij.at[b],
                                  sem_e.at[b]).wait()

        def wait_gather(b):
            pltpu.make_async_copy(x_hbm.at[pl.ds(0, _CHUNK)], rows.at[b],
                                  sem_g.at[b]).wait()

        def wait_scatter(b):
            pltpu.make_async_copy(rows.at[b], acc.at[pl.ds(0, _CHUNK)],
                                  sem_s.at[b]).wait()

        for t in range(_NB):
            @pl.when(t < my_chunks)
            def _(t=t):
                fetch_idx(t, t)

        zv = jnp.zeros((16,), jnp.float32)
        zb = _NB - 1
        zbuf = rows.at[zb, pl.ds(0, zrows)]

        @pl.loop(0, zrows)
        def _(r):
            for j in range(d // 16):
                rows[zb, r, pl.ds(j * 16, 16)] = zv

        nz = rows_per_tile // zrows
        zcopies = [
            pltpu.async_copy(zbuf,
                             acc.at[pl.ds(row0 + j * zrows, zrows)], zsem)
            for j in range(nz)
        ]
        if rem_rows:
            @pl.when(sid == ns - 1)
            def _():
                for j in range(rem_rows // zrows):
                    pltpu.async_copy(
                        zbuf,
                        acc.at[pl.ds(rows_per_tile * ns + j * zrows, zrows)],
                        zsem,
                    ).wait()

        for t in range(2):
            @pl.when(t < my_chunks)
            def _(t=t):
                wait_idx(t)
                issue_gather(t, t)

        for cp in zcopies:
            cp.wait()

        plsc.subcore_barrier()

        num_groups = (my_chunks + _NI - 1) // _NI

        @pl.loop(0, num_groups)
        def _(g):
            t0 = g * _NI
            for k in range(_NI):
                t = t0 + k
                rb = k % _NB
                rb2 = (k + 2) % _NB
                ib = k
                ib2 = (k + 2) % _NI
                ib4 = (k + 4) % _NI

                @pl.when(t + 2 < my_chunks)
                def _(t=t, rb2=rb2, ib2=ib2):
                    if k < 2:
                        @pl.when(t >= 2)
                        def _():
                            wait_scatter(rb2)
                    else:
                        wait_scatter(rb2)
                    wait_idx(ib2)
                    issue_gather(ib2, rb2)

                @pl.when(t < my_chunks)
                def _(t=t, rb=rb, ib=ib):
                    wait_gather(rb)
                    pltpu.async_copy(rows.at[rb], acc.at[ij.at[ib, 1]],
                                     sem_s.at[rb], add=True)

                @pl.when(t + 4 < my_chunks)
                def _(t=t, ib4=ib4):
                    fetch_idx(t + 4, ib4)

        for b in range(_NB):
            @pl.when(b < my_chunks)
            def _(b=b):
                wait_scatter(b)

        plsc.subcore_barrier()

        pltpu.sync_copy(
            acc.at[pl.ds(row0, rows_per_tile)],
            out_hbm.at[cid, pl.ds(row0, rows_per_tile)],
        )
        if rem_rows:
            @pl.when(sid == ns - 1)
            def _():
                pltpu.sync_copy(
                    acc.at[pl.ds(rows_per_tile * ns, rem_rows)],
                    out_hbm.at[cid, pl.ds(rows_per_tile * ns, rem_rows)],
                )

    return seg_sum(edges, x)


def _tc_combine(partials, x, W1, W2, b):
    n, d = x.shape
    bm = 1000
    assert n % bm == 0

    def body(p_ref, x_ref, w1_ref, w2_ref, b_ref, o_ref):
        agg = p_ref[0] + p_ref[1]
        cdims = (((1,), (1,)), ((), ()))
        o_ref[...] = (
            lax.dot_general(agg, w1_ref[...], cdims,
                            preferred_element_type=jnp.float32)
            + lax.dot_general(x_ref[...], w2_ref[...], cdims,
                              preferred_element_type=jnp.float32)
            + b_ref[...]
        )

    return pl.pallas_call(
        body,
        grid=(n // bm,),
        in_specs=[
            pl.BlockSpec((2, bm, d), lambda i: (0, i, 0)),
            pl.BlockSpec((bm, d), lambda i: (i, 0)),
            pl.BlockSpec((d, d), lambda i: (0, 0)),
            pl.BlockSpec((d, d), lambda i: (0, 0)),
            pl.BlockSpec((1, d), lambda i: (0, 0)),
        ],
        out_specs=pl.BlockSpec((bm, d), lambda i: (i, 0)),
        out_shape=jax.ShapeDtypeStruct((n, d), jnp.float32),
    )(partials, x, W1, W2, b)


def kernel(x, edge_index, W1, b1, W2, b2, l):
    e = edge_index.shape[1]
    assert e % _CHUNK == 0
    ep = (edge_index.astype(jnp.int32)
          .reshape(2, e // _CHUNK, _CHUNK)
          .transpose(1, 0, 2))
    partials = _sc_segment_sum(ep, x)
    b = (b1 + b2).reshape(1, -1)
    return _tc_combine(partials, x, W1, W2, b)


# speedup vs baseline: 1.2155x; 1.2155x over previous
import functools

import jax
import jax.numpy as jnp
from jax import lax
from jax.experimental import pallas as pl
from jax.experimental.pallas import tpu as pltpu
from jax.experimental.pallas import tpu_sc as plsc

_CHUNK = 128
_NB = 3
_NI = 6
_PF = 4


def _sc_segment_sum(edges, x):
    n, d = x.shape
    num_chunks = edges.shape[0]
    info = plsc.get_sparse_core_info()
    nc, ns = info.num_cores, info.num_subcores
    nw = nc * ns
    rows_per_tile = (n // ns) // 8 * 8
    rem_rows = n - rows_per_tile * ns
    zrows = 16
    assert rows_per_tile % zrows == 0 and rem_rows % zrows == 0
    assert _CHUNK >= zrows

    mesh = plsc.VectorSubcoreMesh(core_axis_name="c", subcore_axis_name="s")

    @functools.partial(
        pl.kernel,
        out_type=jax.ShapeDtypeStruct((nc, n, d), jnp.float32),
        mesh=mesh,
        scratch_types=[
            pltpu.VMEM_SHARED((n, d), jnp.float32),
            pltpu.VMEM((_NI, 2, _CHUNK), jnp.int32),
            pltpu.VMEM((_NB, _CHUNK, d), jnp.float32),
            pltpu.SemaphoreType.DMA((_NI,)),
            pltpu.SemaphoreType.DMA((_NB,)),
            pltpu.SemaphoreType.DMA((_NB,)),
            pltpu.SemaphoreType.DMA,
        ],
    )
    def seg_sum(edges_hbm, x_hbm, out_hbm, acc, ij, rows,
                sem_e, sem_g, sem_s, zsem):
        cid = lax.axis_index("c")
        sid = lax.axis_index("s")
        wid = sid * nc + cid
        row0 = sid * rows_per_tile

        my_chunks = (num_chunks - wid + nw - 1) // nw

        def fetch_idx(i, b):
            pltpu.async_copy(edges_hbm.at[wid + i * nw], ij.at[b],
                             sem_e.at[b])

        def issue_gather(ib, rb):
            pltpu.async_copy(x_hbm.at[ij.at[ib, 0]], rows.at[rb],
                             sem_g.at[rb])

        def wait_idx(b):
            pltpu.make_async_copy(edges_hbm.at[0], ij.at[b],
                                  sem_e.at[b]).wait()

        def wait_gather(b):
            pltpu.make_async_copy(x_hbm.at[pl.ds(0, _CHUNK)], rows.at[b],
                                  sem_g.at[b]).wait()

        def wait_scatter(b):
            pltpu.make_async_copy(rows.at[b], acc.at[pl.ds(0, _CHUNK)],
                                  sem_s.at[b]).wait()

        for t in range(_PF):
            @pl.when(t < my_chunks)
            def _(t=t):
                fetch_idx(t, t)

        zv = jnp.zeros((16,), jnp.float32)
        zb = _NB - 1
        zbuf = rows.at[zb, pl.ds(0, zrows)]

        @pl.loop(0, zrows)
        def _(r):
            for j in range(d // 16):
                rows[zb, r, pl.ds(j * 16, 16)] = zv

        nz = rows_per_tile // zrows
        zcopies = [
            pltpu.async_copy(zbuf,
                             acc.at[pl.ds(row0 + j * zrows, zrows)], zsem)
            for j in range(nz)
        ]
        if rem_rows:
            @pl.when(sid == ns - 1)
            def _():
                for j in range(rem_rows // zrows):
                    pltpu.async_copy(
                        zbuf,
                        acc.at[pl.ds(rows_per_tile * ns + j * zrows, zrows)],
                        zsem,
                    ).wait()

        for t in range(2):
            @pl.when(t < my_chunks)
            def _(t=t):
                wait_idx(t)
                issue_gather(t, t)

        for cp in zcopies:
            cp.wait()

        plsc.subcore_barrier()

        num_groups = (my_chunks + _NI - 1) // _NI

        @pl.loop(0, num_groups)
        def _(g):
            t0 = g * _NI
            for k in range(_NI):
                t = t0 + k
                rb = k % _NB
                rb2 = (k + 2) % _NB
                ib = k
                ib2 = (k + 2) % _NI
                ibf = (k + _PF) % _NI

                @pl.when(t + 2 < my_chunks)
                def _(t=t, rb2=rb2, ib2=ib2):
                    if k < _NB - 2:
                        @pl.when(t >= _NB - 2)
                        def _():
                            wait_scatter(rb2)
                    else:
                        wait_scatter(rb2)
                    wait_idx(ib2)
                    issue_gather(ib2, rb2)

                @pl.when(t < my_chunks)
                def _(t=t, rb=rb, ib=ib):
                    wait_gather(rb)
                    pltpu.async_copy(rows.at[rb], acc.at[ij.at[ib, 1]],
                                     sem_s.at[rb], add=True)

                @pl.when(t + _PF < my_chunks)
                def _(t=t, ibf=ibf):
                    fetch_idx(t + _PF, ibf)

        for b in range(_NB):
            @pl.when(b < my_chunks)
            def _(b=b):
                wait_scatter(b)

        plsc.subcore_barrier()

        pltpu.sync_copy(
            acc.at[pl.ds(row0, rows_per_tile)],
            out_hbm.at[cid, pl.ds(row0, rows_per_tile)],
        )
        if rem_rows:
            @pl.when(sid == ns - 1)
            def _():
                pltpu.sync_copy(
                    acc.at[pl.ds(rows_per_tile * ns, rem_rows)],
                    out_hbm.at[cid, pl.ds(rows_per_tile * ns, rem_rows)],
                )

    return seg_sum(edges, x)


def _tc_combine(partials, x, W1, W2, b):
    n, d = x.shape
    bm = 1000
    assert n % bm == 0

    def body(p_ref, x_ref, w1_ref, w2_ref, b_ref, o_ref):
        agg = p_ref[0] + p_ref[1]
        cdims = (((1,), (1,)), ((), ()))
        o_ref[...] = (
            lax.dot_general(agg, w1_ref[...], cdims,
                            preferred_element_type=jnp.float32)
            + lax.dot_general(x_ref[...], w2_ref[...], cdims,
                              preferred_element_type=jnp.float32)
            + b_ref[...]
        )

    return pl.pallas_call(
        body,
        grid=(n // bm,),
        in_specs=[
            pl.BlockSpec((2, bm, d), lambda i: (0, i, 0)),
            pl.BlockSpec((bm, d), lambda i: (i, 0)),
            pl.BlockSpec((d, d), lambda i: (0, 0)),
            pl.BlockSpec((d, d), lambda i: (0, 0)),
            pl.BlockSpec((1, d), lambda i: (0, 0)),
        ],
        out_specs=pl.BlockSpec((bm, d), lambda i: (i, 0)),
        out_shape=jax.ShapeDtypeStruct((n, d), jnp.float32),
    )(partials, x, W1, W2, b)


def kernel(x, edge_index, W1, b1, W2, b2, l):
    e = edge_index.shape[1]
    assert e % _CHUNK == 0
    ep = (edge_index.astype(jnp.int32)
          .reshape(2, e // _CHUNK, _CHUNK)
          .transpose(1, 0, 2))
    partials = _sc_segment_sum(ep, x)
    b = (b1 + b2).reshape(1, -1)
    return _tc_combine(partials, x, W1, W2, b)
